# baseline (speedup 1.0000x reference)
import jax
import jax.numpy as jnp
import numpy as np
from jax.experimental import pallas as pl
from jax.experimental.pallas import tpu as pltpu

_B, _N, _D, _H = 2, 1024, 128, 8
_F = _D // _H
_C = 4
_NB = _N // _C


def _gat_dense_kernel(x_ref, adj_ref, w_ref, asrc_ref, adst_ref, bias_ref,
                      out_ref, xp_ref, xpbf_ref, asrc2_ref, adstT2_ref,
                      mbias_ref):
    b = pl.program_id(0)
    c = pl.program_id(1)

    @pl.when(c == 0)
    def _project():
        x = x_ref[0]
        xp = jnp.dot(x, w_ref[...], preferred_element_type=jnp.float32)
        xp_ref[...] = xp
        xpbf_ref[...] = xp.astype(jnp.bfloat16)
        asrc2_ref[...] = jnp.dot(xp, asrc_ref[...],
                                 preferred_element_type=jnp.float32)
        adstT2_ref[...] = jax.lax.dot_general(
            adst_ref[...], xp,
            dimension_numbers=(((0,), (1,)), ((), ())),
            preferred_element_type=jnp.float32)

    @pl.when(b == 0)
    def _build_mask():
        adj = adj_ref[...]
        row = jax.lax.broadcasted_iota(jnp.int32, (_N, _NB), 0)
        col = jax.lax.broadcasted_iota(jnp.int32, (_N, _NB), 1) + c * _NB
        valid = (adj != 0) | (row == col)
        mbias_ref[:, pl.ds(c * _NB, _NB)] = jnp.where(valid, 0.0, -1e30)

    a_src2 = asrc2_ref[...]
    mbias = mbias_ref[:, pl.ds(c * _NB, _NB)]
    xp_bf = xpbf_ref[...]
    ones_col = jnp.ones((_N, 1), dtype=jnp.bfloat16)
    for h in range(_H):
        src_h = a_src2[:, h:h + 1]
        dst_h = adstT2_ref[h:h + 1, pl.ds(c * _NB, _NB)]
        alpha = src_h + dst_h
        alpha = jnp.maximum(alpha, 0.2 * alpha)
        ex = jnp.exp2(alpha + mbias)
        ex_bf = ex.astype(jnp.bfloat16)

        xp_h = xp_bf[:, h * _F:(h + 1) * _F]
        out_h = jax.lax.dot_general(
            ex_bf, xp_h,
            dimension_numbers=(((0,), (0,)), ((), ())),
            preferred_element_type=jnp.float32)
        denom = jax.lax.dot_general(
            ex_bf, ones_col,
            dimension_numbers=(((0,), (0,)), ((), ())),
            preferred_element_type=jnp.float32)
        scale = 1.0 / (denom + 1e-16)
        out_ref[0, :, h * _F:(h + 1) * _F] = (
            out_h * scale + bias_ref[0, h * _F:(h + 1) * _F])


def kernel(hidden_states, adjacency_matrix, W, att_src, att_dst, bias):
    H, F, D, N, B = _H, _F, _D, _N, _B
    log2e = np.float32(np.log2(np.e))
    eye_h = jnp.eye(H, dtype=jnp.float32)
    a_src_mat = (att_src[:, :, None] * eye_h[:, None, :]).reshape(D, H) * log2e
    a_dst_mat = (att_dst[:, :, None] * eye_h[:, None, :]).reshape(D, H) * log2e
    bias2 = bias.reshape(1, D)

    out = pl.pallas_call(
        _gat_dense_kernel,
        grid=(B, _C),
        in_specs=[
            pl.BlockSpec((1, N, D), lambda b, c: (b, 0, 0)),
            pl.BlockSpec((N, _NB), lambda b, c: (0, c)),
            pl.BlockSpec((D, D), lambda b, c: (0, 0)),
            pl.BlockSpec((D, H), lambda b, c: (0, 0)),
            pl.BlockSpec((D, H), lambda b, c: (0, 0)),
            pl.BlockSpec((1, D), lambda b, c: (0, 0)),
        ],
        out_specs=pl.BlockSpec((1, _NB, D), lambda b, c: (b, c, 0)),
        out_shape=jax.ShapeDtypeStruct((B, N, D), jnp.float32),
        scratch_shapes=[
            pltpu.VMEM((N, D), jnp.float32),
            pltpu.VMEM((N, D), jnp.bfloat16),
            pltpu.VMEM((N, H), jnp.float32),
            pltpu.VMEM((H, N), jnp.float32),
            pltpu.VMEM((N, N), jnp.float32),
        ],
        compiler_params=pltpu.CompilerParams(
            dimension_semantics=("arbitrary", "arbitrary")),
    )(hidden_states, adjacency_matrix, W, a_src_mat, a_dst_mat, bias2)
    return out


# speedup vs baseline: 2.3594x; 2.3594x over previous
import jax
import jax.numpy as jnp
import numpy as np
from jax.experimental import pallas as pl
from jax.experimental.pallas import tpu as pltpu

_B, _N, _D, _H = 2, 1024, 128, 8
_F = _D // _H


def _gat_dense_kernel(x_ref, adj_ref, w_ref, asrc_ref, adst_ref, bias_ref,
                      out_ref, mbias_ref):
    adj = adj_ref[...]
    row = jax.lax.broadcasted_iota(jnp.int32, (_N, _N), 0)
    col = jax.lax.broadcasted_iota(jnp.int32, (_N, _N), 1)
    diag = row == col
    valid = ((adj != 0) & jnp.logical_not(diag)) | diag
    mbias_ref[...] = jnp.where(valid, 0.0, -1e30)
    mbias = mbias_ref[...]

    w = w_ref[...]
    ones_col = jnp.ones((_N, 1), dtype=jnp.bfloat16)

    for b in range(_B):
        x = x_ref[b]
        xp = jnp.dot(x, w, preferred_element_type=jnp.float32)

        a_src2 = jnp.dot(xp, asrc_ref[...],
                         preferred_element_type=jnp.float32)
        a_dstT2 = jax.lax.dot_general(
            adst_ref[...], xp,
            dimension_numbers=(((0,), (1,)), ((), ())),
            preferred_element_type=jnp.float32)

        xp_bf = xp.astype(jnp.bfloat16)
        for h in range(_H):
            src_h = a_src2[:, h:h + 1]
            dst_h = a_dstT2[h:h + 1, :]
            alpha = src_h + dst_h
            alpha = jnp.maximum(alpha, 0.2 * alpha)
            ex = jnp.exp2(alpha + mbias)
            ex_bf = ex.astype(jnp.bfloat16)

            xp_h = xp_bf[:, h * _F:(h + 1) * _F]
            out_h = jax.lax.dot_general(
                ex_bf, xp_h,
                dimension_numbers=(((0,), (0,)), ((), ())),
                preferred_element_type=jnp.float32)
            denom = jax.lax.dot_general(
                ex_bf, ones_col,
                dimension_numbers=(((0,), (0,)), ((), ())),
                preferred_element_type=jnp.float32)
            scale = 1.0 / (denom + 1e-16)
            out_ref[b, :, h * _F:(h + 1) * _F] = (
                out_h * scale + bias_ref[0, h * _F:(h + 1) * _F])


def kernel(hidden_states, adjacency_matrix, W, att_src, att_dst, bias):
    H, F, D, N, B = _H, _F, _D, _N, _B
    log2e = np.float32(np.log2(np.e))
    eye_h = jnp.eye(H, dtype=jnp.float32)
    a_src_mat = (att_src[:, :, None] * eye_h[:, None, :]).reshape(D, H) * log2e
    a_dst_mat = (att_dst[:, :, None] * eye_h[:, None, :]).reshape(D, H) * log2e
    bias2 = bias.reshape(1, D)

    out = pl.pallas_call(
        _gat_dense_kernel,
        out_shape=jax.ShapeDtypeStruct((B, N, D), jnp.float32),
        scratch_shapes=[pltpu.VMEM((N, N), jnp.float32)],
    )(hidden_states, adjacency_matrix, W, a_src_mat, a_dst_mat, bias2)
    return out


# speedup vs baseline: 2.5468x; 1.0794x over previous
import jax
import jax.numpy as jnp
import numpy as np
from jax.experimental import pallas as pl
from jax.experimental.pallas import tpu as pltpu

_B, _N, _D, _H = 2, 1024, 128, 8
_F = _D // _H


def _gat_dense_kernel(x_ref, adj_ref, w_ref, asrc_ref, adst_ref, bias_ref,
                      out_ref, mbias_ref):
    x = x_ref[0]
    w = w_ref[...]
    xp = jnp.dot(x, w, preferred_element_type=jnp.float32)

    a_dst2 = jnp.dot(xp, adst_ref[...], preferred_element_type=jnp.float32)
    a_srcT2 = jax.lax.dot_general(
        asrc_ref[...], xp,
        dimension_numbers=(((0,), (1,)), ((), ())),
        preferred_element_type=jnp.float32)

    adjT = jnp.transpose(adj_ref[...])
    row = jax.lax.broadcasted_iota(jnp.int32, (_N, _N), 0)
    col = jax.lax.broadcasted_iota(jnp.int32, (_N, _N), 1)
    diag = row == col
    valid = ((adjT != 0) & jnp.logical_not(diag)) | diag
    mbias_ref[...] = jnp.where(valid, 0.0, -1e30)

    ones_col = jnp.ones((_N, 1), dtype=jnp.bfloat16)
    mbias = mbias_ref[...]
    xp_bf = xp.astype(jnp.bfloat16)
    for h in range(_H):
        dst_h = a_dst2[:, h:h + 1]
        src_h = a_srcT2[h:h + 1, :]
        alpha = dst_h + src_h
        alpha = jnp.maximum(alpha, 0.2 * alpha)
        exT = jnp.exp2(alpha + mbias)
        exT_bf = exT.astype(jnp.bfloat16)

        xp_h = xp_bf[:, h * _F:(h + 1) * _F]
        out_h = jax.lax.dot_general(
            exT_bf, xp_h,
            dimension_numbers=(((1,), (0,)), ((), ())),
            preferred_element_type=jnp.float32)
        denom = jax.lax.dot_general(
            exT_bf, ones_col,
            dimension_numbers=(((1,), (0,)), ((), ())),
            preferred_element_type=jnp.float32)
        scale = 1.0 / (denom + 1e-16)
        out_ref[0, :, h * _F:(h + 1) * _F] = (
            out_h * scale + bias_ref[0, h * _F:(h + 1) * _F])


def kernel(hidden_states, adjacency_matrix, W, att_src, att_dst, bias):
    H, F, D, N, B = _H, _F, _D, _N, _B
    log2e = np.float32(np.log2(np.e))
    eye_h = jnp.eye(H, dtype=jnp.float32)
    a_src_mat = (att_src[:, :, None] * eye_h[:, None, :]).reshape(D, H) * log2e
    a_dst_mat = (att_dst[:, :, None] * eye_h[:, None, :]).reshape(D, H) * log2e
    bias2 = bias.reshape(1, D)

    out = pl.pallas_call(
        _gat_dense_kernel,
        grid=(B,),
        in_specs=[
            pl.BlockSpec((1, N, D), lambda b: (b, 0, 0)),
            pl.BlockSpec((N, N), lambda b: (0, 0)),
            pl.BlockSpec((D, D), lambda b: (0, 0)),
            pl.BlockSpec((D, H), lambda b: (0, 0)),
            pl.BlockSpec((D, H), lambda b: (0, 0)),
            pl.BlockSpec((1, D), lambda b: (0, 0)),
        ],
        out_specs=pl.BlockSpec((1, N, D), lambda b: (b, 0, 0)),
        out_shape=jax.ShapeDtypeStruct((B, N, D), jnp.float32),
        scratch_shapes=[pltpu.VMEM((N, N), jnp.float32)],
        compiler_params=pltpu.CompilerParams(
            dimension_semantics=("arbitrary",)),
    )(hidden_states, adjacency_matrix, W, a_src_mat, a_dst_mat, bias2)
    return out
